# Initial kernel scaffold; baseline (speedup 1.0000x reference)
import jax
import jax.numpy as jnp
from jax.experimental import pallas as pl


def kernel(x, edge_index, graph_indices, W1l, W1r, b1, W2l, W2r, b2, Wal, War, ba, Wv, bv):
    raise NotImplementedError("write your pallas kernel here")


# speedup vs baseline: 5.8140x; 5.8140x over previous
import functools

import jax
import jax.numpy as jnp
from jax import lax
from jax.experimental import pallas as pl
from jax.experimental.pallas import tpu as pltpu
from jax.experimental.pallas import tpu_sc as plsc

N, E, D, G = 10000, 320000, 128, 64
NC, NS = 2, 16
NW = NC * NS
EPW = E // NW
KC = 80
NCHUNK = EPW // KC
NPAD = 10240
RPT = NPAD // NS
RB = 1000
NRB = N // RB

_f32 = jnp.float32
_mesh = plsc.VectorSubcoreMesh(core_axis_name="c", subcore_axis_name="s")
_sc_params = pltpu.CompilerParams(needs_layout_passes=False)


def _seg_body(with_deg, x_hbm, src_hbm, dst_hbm, z2d_hbm, zn_hbm, *refs):
    if with_deg:
        (agg_out, deg_out, src_v, dst_v, rows_v, deg_v, accum, sem) = refs
    else:
        (agg_out, src_v, dst_v, rows_v, accum, sem) = refs
    c = lax.axis_index("c")
    s = lax.axis_index("s")
    wid = s * NC + c

    pltpu.sync_copy(z2d_hbm, accum.at[pl.ds(s * RPT, RPT)])
    if with_deg:
        pltpu.sync_copy(zn_hbm, deg_v)
    plsc.subcore_barrier()

    ones16 = jnp.ones((16,), _f32)
    base_e = wid * EPW

    def chunk(i, carry):
        off = base_e + i * KC
        pltpu.sync_copy(src_hbm.at[pl.ds(off, KC)], src_v)
        pltpu.sync_copy(dst_hbm.at[pl.ds(off, KC)], dst_v)
        pltpu.async_copy(x_hbm.at[src_v], rows_v, sem).wait()
        pltpu.sync_copy(rows_v, accum.at[dst_v], add=True)
        if with_deg:
            for v in range(KC // 16):
                idx = dst_v[pl.ds(v * 16, 16)]
                plsc.addupdate_scatter(deg_v, [idx], ones16)
        return carry

    lax.fori_loop(0, NCHUNK, chunk, 0)
    plsc.subcore_barrier()

    pltpu.sync_copy(accum.at[pl.ds(s * RPT, RPT)],
                    agg_out.at[c, pl.ds(s * RPT, RPT)])
    if with_deg:
        pltpu.sync_copy(deg_v, deg_out.at[wid])


def _sc_seg_with_deg(x, src, dst, z2d, zn):
    out_type = (jax.ShapeDtypeStruct((NC, NPAD, D), _f32),
                jax.ShapeDtypeStruct((NW, N), _f32))
    scratch = [
        pltpu.VMEM((KC,), jnp.int32),
        pltpu.VMEM((KC,), jnp.int32),
        pltpu.VMEM((KC, D), _f32),
        pltpu.VMEM((N,), _f32),
        pltpu.VMEM_SHARED((NPAD, D), _f32),
        pltpu.SemaphoreType.DMA,
    ]
    f = pl.kernel(functools.partial(_seg_body, True), out_type=out_type,
                  mesh=_mesh, scratch_types=scratch, compiler_params=_sc_params)
    return f(x, src, dst, z2d, zn)


def _sc_seg(x, src, dst, z2d, zn):
    out_type = jax.ShapeDtypeStruct((NC, NPAD, D), _f32)
    scratch = [
        pltpu.VMEM((KC,), jnp.int32),
        pltpu.VMEM((KC,), jnp.int32),
        pltpu.VMEM((KC, D), _f32),
        pltpu.VMEM_SHARED((NPAD, D), _f32),
        pltpu.SemaphoreType.DMA,
    ]
    f = pl.kernel(functools.partial(_seg_body, False), out_type=out_type,
                  mesh=_mesh, scratch_types=scratch, compiler_params=_sc_params)
    return f(x, src, dst, z2d, zn)


def _adv_body(a_hbm, src_hbm, dst_hbm, zn_hbm, out_hbm,
              a_v, accum_v, src_v, dst_v, sem):
    c = lax.axis_index("c")
    s = lax.axis_index("s")
    wid = s * NC + c
    pltpu.sync_copy(a_hbm, a_v)
    pltpu.sync_copy(zn_hbm, accum_v)
    base_e = wid * EPW

    def chunk(i, carry):
        off = base_e + i * KC
        pltpu.sync_copy(src_hbm.at[pl.ds(off, KC)], src_v)
        pltpu.sync_copy(dst_hbm.at[pl.ds(off, KC)], dst_v)
        for v in range(KC // 16):
            sidx = src_v[pl.ds(v * 16, 16)]
            vals = plsc.load_gather(a_v, [sidx])
            didx = dst_v[pl.ds(v * 16, 16)]
            plsc.addupdate_scatter(accum_v, [didx], vals)
        return carry

    lax.fori_loop(0, NCHUNK, chunk, 0)
    pltpu.sync_copy(accum_v, out_hbm.at[wid])


def _sc_adv(a, src, dst, zn):
    out_type = jax.ShapeDtypeStruct((NW, N), _f32)
    scratch = [
        pltpu.VMEM((N,), _f32),
        pltpu.VMEM((N,), _f32),
        pltpu.VMEM((KC,), jnp.int32),
        pltpu.VMEM((KC,), jnp.int32),
        pltpu.SemaphoreType.DMA,
    ]
    f = pl.kernel(_adv_body, out_type=out_type, mesh=_mesh,
                  scratch_types=scratch, compiler_params=_sc_params)
    return f(a, src, dst, zn)


def _dot(a, b):
    return lax.dot_general(a, b, (((1,), (0,)), ((), ())),
                           precision=lax.Precision.HIGHEST,
                           preferred_element_type=_f32)


def _inv_deg(degp):
    deg = jnp.sum(degp.reshape(NW, RB), axis=0)
    return jnp.where(deg > 0, 1.0 / jnp.maximum(deg, 1.0), 0.0)


def _tc1_body(aggp_ref, x_ref, degp_ref, wl_ref, wr_ref, b_ref, h_ref):
    inv = _inv_deg(degp_ref[...])
    mean = (aggp_ref[0] + aggp_ref[1]) * inv[:, None]
    h = _dot(mean, wl_ref[...]) + _dot(x_ref[...], wr_ref[...]) + b_ref[...]
    h_ref[...] = jnp.maximum(h, 0.0)


def _tc1(aggp, x, degp4, wlT, wrT, b):
    return pl.pallas_call(
        _tc1_body,
        grid=(NRB,),
        in_specs=[
            pl.BlockSpec((NC, RB, D), lambda i: (0, i, 0)),
            pl.BlockSpec((RB, D), lambda i: (i, 0)),
            pl.BlockSpec((NW, 1, 1, RB), lambda i: (0, i, 0, 0)),
            pl.BlockSpec((D, D), lambda i: (0, 0)),
            pl.BlockSpec((D, D), lambda i: (0, 0)),
            pl.BlockSpec((1, D), lambda i: (0, 0)),
        ],
        out_specs=pl.BlockSpec((RB, D), lambda i: (i, 0)),
        out_shape=jax.ShapeDtypeStruct((N, D), _f32),
    )(aggp, x, degp4, wlT, wrT, b)


def _one_hot(gi_ref):
    gi = gi_ref[...].reshape(RB)
    return (gi[:, None] == lax.broadcasted_iota(jnp.int32, (RB, G), 1)
            ).astype(_f32)


def _tc2_body(aggp_ref, h_ref, degp_ref, wl_ref, wr_ref, b_ref,
              wa_ref, wr2_ref, gi_ref, a_ref, r_ref, gp_ref):
    i = pl.program_id(0)
    inv = _inv_deg(degp_ref[...])
    mean = (aggp_ref[0] + aggp_ref[1]) * inv[:, None]
    emb = _dot(mean, wl_ref[...]) + _dot(h_ref[...], wr_ref[...]) + b_ref[...]
    a_ref[...] = jnp.sum(emb * wa_ref[...], axis=1).reshape(1, 1, RB)
    r_ref[...] = jnp.sum(emb * wr2_ref[...], axis=1).reshape(1, 1, RB)
    oh = _one_hot(gi_ref)

    @pl.when(i == 0)
    def _():
        gp_ref[...] = jnp.zeros_like(gp_ref)

    gp_ref[...] += lax.dot_general(oh, emb, (((0,), (0,)), ((), ())),
                                   precision=lax.Precision.HIGHEST,
                                   preferred_element_type=_f32)


def _tc2(aggp, h, degp4, wlT, wrT, b, wal, war, gi3):
    return pl.pallas_call(
        _tc2_body,
        grid=(NRB,),
        in_specs=[
            pl.BlockSpec((NC, RB, D), lambda i: (0, i, 0)),
            pl.BlockSpec((RB, D), lambda i: (i, 0)),
            pl.BlockSpec((NW, 1, 1, RB), lambda i: (0, i, 0, 0)),
            pl.BlockSpec((D, D), lambda i: (0, 0)),
            pl.BlockSpec((D, D), lambda i: (0, 0)),
            pl.BlockSpec((1, D), lambda i: (0, 0)),
            pl.BlockSpec((1, D), lambda i: (0, 0)),
            pl.BlockSpec((1, D), lambda i: (0, 0)),
            pl.BlockSpec((1, 1, RB), lambda i: (i, 0, 0)),
        ],
        out_specs=[
            pl.BlockSpec((1, 1, RB), lambda i: (i, 0, 0)),
            pl.BlockSpec((1, 1, RB), lambda i: (i, 0, 0)),
            pl.BlockSpec((G, D), lambda i: (0, 0)),
        ],
        out_shape=[
            jax.ShapeDtypeStruct((NRB, 1, RB), _f32),
            jax.ShapeDtypeStruct((NRB, 1, RB), _f32),
            jax.ShapeDtypeStruct((G, D), _f32),
        ],
    )(aggp, h, degp4, wlT, wrT, b, wal, war, gi3)


def _tc3a_body(advp_ref, degp_ref, r_ref, gp_ref, wv_ref, ba_ref, bv_ref,
               gi_ref, adv_ref, asum_ref, cnt_ref, val_ref):
    i = pl.program_id(0)
    inv = _inv_deg(degp_ref[...])
    agg_a = jnp.sum(advp_ref[...].reshape(NW, RB), axis=0)
    r = r_ref[...].reshape(RB)
    adv = 2.0 * jnp.tanh(agg_a * inv + r + ba_ref[0, 0])
    adv_ref[...] = adv.reshape(1, 1, RB)
    oh = _one_hot(gi_ref)

    @pl.when(i == 0)
    def _():
        asum_ref[...] = jnp.zeros_like(asum_ref)
        cnt_ref[...] = jnp.zeros_like(cnt_ref)
        gp = gp_ref[...]
        val_ref[...] = jnp.tanh(
            jnp.sum(gp * wv_ref[...], axis=1) + bv_ref[0, 0]).reshape(1, G)

    asum_ref[...] += jnp.sum(oh * adv[:, None], axis=0).reshape(1, G)
    cnt_ref[...] += jnp.sum(oh, axis=0).reshape(1, G)


def _tc3a(advp4, degp4, r3, gp, wv, ba, bv, gi3):
    return pl.pallas_call(
        _tc3a_body,
        grid=(NRB,),
        in_specs=[
            pl.BlockSpec((NW, 1, 1, RB), lambda i: (0, i, 0, 0)),
            pl.BlockSpec((NW, 1, 1, RB), lambda i: (0, i, 0, 0)),
            pl.BlockSpec((1, 1, RB), lambda i: (i, 0, 0)),
            pl.BlockSpec((G, D), lambda i: (0, 0)),
            pl.BlockSpec((1, D), lambda i: (0, 0)),
            pl.BlockSpec((1, 1), lambda i: (0, 0)),
            pl.BlockSpec((1, 1), lambda i: (0, 0)),
            pl.BlockSpec((1, 1, RB), lambda i: (i, 0, 0)),
        ],
        out_specs=[
            pl.BlockSpec((1, 1, RB), lambda i: (i, 0, 0)),
            pl.BlockSpec((1, G), lambda i: (0, 0)),
            pl.BlockSpec((1, G), lambda i: (0, 0)),
            pl.BlockSpec((1, G), lambda i: (0, 0)),
        ],
        out_shape=[
            jax.ShapeDtypeStruct((NRB, 1, RB), _f32),
            jax.ShapeDtypeStruct((1, G), _f32),
            jax.ShapeDtypeStruct((1, G), _f32),
            jax.ShapeDtypeStruct((1, G), _f32),
        ],
    )(advp4, degp4, r3, gp, wv, ba, bv, gi3)


def _tc3b_body(adv_ref, val_ref, asum_ref, cnt_ref, gi_ref, out_ref):
    cnt = cnt_ref[...].reshape(G)
    asum = asum_ref[...].reshape(G)
    amean = jnp.where(cnt > 0, asum / jnp.maximum(cnt, 1.0), 0.0)
    corr = val_ref[...].reshape(G) - amean
    oh = _one_hot(gi_ref)
    per_row = jnp.sum(oh * corr[None, :], axis=1)
    out_ref[...] = jnp.tanh(per_row + adv_ref[...].reshape(RB)
                            ).reshape(1, 1, RB)


def _tc3b(adv3, val, asum, cnt, gi3):
    return pl.pallas_call(
        _tc3b_body,
        grid=(NRB,),
        in_specs=[
            pl.BlockSpec((1, 1, RB), lambda i: (i, 0, 0)),
            pl.BlockSpec((1, G), lambda i: (0, 0)),
            pl.BlockSpec((1, G), lambda i: (0, 0)),
            pl.BlockSpec((1, G), lambda i: (0, 0)),
            pl.BlockSpec((1, 1, RB), lambda i: (i, 0, 0)),
        ],
        out_specs=pl.BlockSpec((1, 1, RB), lambda i: (i, 0, 0)),
        out_shape=jax.ShapeDtypeStruct((NRB, 1, RB), _f32),
    )(adv3, val, asum, cnt, gi3)


def kernel(x, edge_index, graph_indices, W1l, W1r, b1, W2l, W2r, b2,
           Wal, War, ba, Wv, bv):
    src = edge_index[0]
    dst = edge_index[1]
    z2d = jnp.zeros((RPT, D), _f32)
    zn = jnp.zeros((N,), _f32)
    gi3 = graph_indices.reshape(NRB, 1, RB)

    agg1p, degp = _sc_seg_with_deg(x, src, dst, z2d, zn)
    degp4 = degp.reshape(NW, NRB, 1, RB)
    h = _tc1(agg1p, x, degp4, W1l.T, W1r.T, b1.reshape(1, D))
    agg2p = _sc_seg(h, src, dst, z2d, zn)
    a3, r3, gp = _tc2(agg2p, h, degp4, W2l.T, W2r.T, b2.reshape(1, D),
                      Wal, War, gi3)
    advp = _sc_adv(a3.reshape(N), src, dst, zn)
    adv3, asum, cnt, val = _tc3a(advp.reshape(NW, NRB, 1, RB), degp4, r3, gp,
                                 Wv, ba.reshape(1, 1), bv.reshape(1, 1), gi3)
    out3 = _tc3b(adv3, val, asum, cnt, gi3)
    return out3.reshape(N)


# speedup vs baseline: 11.1101x; 1.9109x over previous
import functools

import jax
import jax.numpy as jnp
from jax import lax
from jax.experimental import pallas as pl
from jax.experimental.pallas import tpu as pltpu
from jax.experimental.pallas import tpu_sc as plsc

N, E, D, G = 10000, 320000, 128, 64
NC, NS = 2, 16
NW = NC * NS
EPW = E // NW
KC = 80
NCHUNK = EPW // KC
NPAD = 10240
RPT = NPAD // NS
RB = 1000
NRB = N // RB

_f32 = jnp.float32
_mesh = plsc.VectorSubcoreMesh(core_axis_name="c", subcore_axis_name="s")
_sc_params = pltpu.CompilerParams(needs_layout_passes=False)


def _seg_body(with_deg, x_hbm, src_hbm, dst_hbm, z2d_hbm, zn_hbm, *refs):
    if with_deg:
        (agg_out, deg_out, src1_v, dstb0, dstb1, deg_v,
         rows0, rows1, accum, sg0, sg1, sa0, sa1, sd0, sd1) = refs
    else:
        (agg_out, src1_v, dstb0, dstb1, rows0, rows1, accum,
         sg0, sg1, sa0, sa1, sd0, sd1) = refs
        deg_v = None
    c = lax.axis_index("c")
    s = lax.axis_index("s")
    wid = s * NC + c
    base_e = wid * EPW

    pltpu.sync_copy(z2d_hbm, accum.at[pl.ds(s * RPT, RPT)])
    pltpu.sync_copy(src_hbm.at[pl.ds(base_e, EPW)], src1_v)
    if with_deg:
        pltpu.sync_copy(zn_hbm, deg_v)
    plsc.subcore_barrier()

    ones16 = jnp.ones((16,), _f32)
    bufs = ((rows0, sg0, sa0, dstb0, sd0), (rows1, sg1, sa1, dstb1, sd1))

    def do_deg(dstb):
        if with_deg:
            for v in range(KC // 16):
                idx = dstb[pl.ds(v * 16, 16)]
                plsc.addupdate_scatter(deg_v, [idx], ones16)

    pltpu.async_copy(dst_hbm.at[pl.ds(base_e, KC)], dstb0, sd0)
    pltpu.async_copy(dst_hbm.at[pl.ds(base_e + KC, KC)], dstb1, sd1)
    pltpu.async_copy(x_hbm.at[src1_v.at[pl.ds(0, KC)]], rows0, sg0)

    def pair(j, carry):
        for b in (0, 1):
            i = 2 * j + b
            rows, sg, sa, dstb, sd = bufs[b]
            o_rows, o_sg, _, _, _ = bufs[1 - b]
            pltpu.make_async_copy(x_hbm.at[pl.ds(0, KC)], rows, sg).wait()
            pltpu.async_copy(
                x_hbm.at[src1_v.at[pl.ds((i + 1) * KC, KC)]], o_rows, o_sg)
            pltpu.make_async_copy(dst_hbm.at[pl.ds(0, KC)], dstb, sd).wait()
            d = pltpu.async_copy(rows, accum.at[dstb], sa, add=True)
            do_deg(dstb)
            d.wait()

            @pl.when(i + 2 < NCHUNK)
            def _():
                pltpu.async_copy(
                    dst_hbm.at[pl.ds(base_e + (i + 2) * KC, KC)], dstb, sd)
        return carry

    lax.fori_loop(0, (NCHUNK - 1) // 2, pair, 0)
    pltpu.make_async_copy(x_hbm.at[pl.ds(0, KC)], rows0, sg0).wait()
    pltpu.make_async_copy(dst_hbm.at[pl.ds(0, KC)], dstb0, sd0).wait()
    pltpu.sync_copy(rows0, accum.at[dstb0], add=True)
    do_deg(dstb0)
    plsc.subcore_barrier()

    pltpu.sync_copy(accum.at[pl.ds(s * RPT, RPT)],
                    agg_out.at[c, pl.ds(s * RPT, RPT)])
    if with_deg:
        pltpu.sync_copy(deg_v, deg_out.at[wid])


_SEG_SEMS = [pltpu.SemaphoreType.DMA] * 6


def _sc_seg_with_deg(x, src, dst, z2d, zn):
    out_type = (jax.ShapeDtypeStruct((NC, NPAD, D), _f32),
                jax.ShapeDtypeStruct((NW, N), _f32))
    scratch = [
        pltpu.VMEM((EPW,), jnp.int32),
        pltpu.VMEM((KC,), jnp.int32),
        pltpu.VMEM((KC,), jnp.int32),
        pltpu.VMEM((N,), _f32),
        pltpu.VMEM((KC, D), _f32),
        pltpu.VMEM((KC, D), _f32),
        pltpu.VMEM_SHARED((NPAD, D), _f32),
    ] + _SEG_SEMS
    f = pl.kernel(functools.partial(_seg_body, True), out_type=out_type,
                  mesh=_mesh, scratch_types=scratch, compiler_params=_sc_params)
    return f(x, src, dst, z2d, zn)


def _seg_body_nodeg(x_hbm, src_hbm, dst_hbm, z2d_hbm, *refs):
    _seg_body(False, x_hbm, src_hbm, dst_hbm, z2d_hbm, None, *refs)


def _sc_seg(x, src, dst, z2d):
    out_type = jax.ShapeDtypeStruct((NC, NPAD, D), _f32)
    scratch = [
        pltpu.VMEM((EPW,), jnp.int32),
        pltpu.VMEM((KC,), jnp.int32),
        pltpu.VMEM((KC,), jnp.int32),
        pltpu.VMEM((KC, D), _f32),
        pltpu.VMEM((KC, D), _f32),
        pltpu.VMEM_SHARED((NPAD, D), _f32),
    ] + _SEG_SEMS
    f = pl.kernel(_seg_body_nodeg, out_type=out_type,
                  mesh=_mesh, scratch_types=scratch, compiler_params=_sc_params)
    return f(x, src, dst, z2d)


_NG = EPW // 16
_ADV_UNROLL = 5


def _adv_body(a_hbm, src_hbm, dst_hbm, zn_hbm, out_hbm,
              a_v, accum_v, src1_v, dst1_v):
    c = lax.axis_index("c")
    s = lax.axis_index("s")
    wid = s * NC + c
    pltpu.sync_copy(a_hbm, a_v)
    pltpu.sync_copy(zn_hbm, accum_v)
    pltpu.sync_copy(src_hbm.at[pl.ds(wid * EPW, EPW)], src1_v)
    pltpu.sync_copy(dst_hbm.at[pl.ds(wid * EPW, EPW)], dst1_v)

    def group(j, carry):
        for v in range(_ADV_UNROLL):
            off = (j * _ADV_UNROLL + v) * 16
            vals = plsc.load_gather(a_v, [src1_v[pl.ds(off, 16)]])
            plsc.addupdate_scatter(accum_v, [dst1_v[pl.ds(off, 16)]], vals)
        return carry

    lax.fori_loop(0, _NG // _ADV_UNROLL, group, 0)
    pltpu.sync_copy(accum_v, out_hbm.at[wid])


def _sc_adv(a, src, dst, zn):
    out_type = jax.ShapeDtypeStruct((NW, N), _f32)
    scratch = [
        pltpu.VMEM((N,), _f32),
        pltpu.VMEM((N,), _f32),
        pltpu.VMEM((EPW,), jnp.int32),
        pltpu.VMEM((EPW,), jnp.int32),
    ]
    f = pl.kernel(_adv_body, out_type=out_type, mesh=_mesh,
                  scratch_types=scratch, compiler_params=_sc_params)
    return f(a, src, dst, zn)


def _dot(a, b):
    return lax.dot_general(a, b, (((1,), (0,)), ((), ())),
                           precision=lax.Precision.HIGHEST,
                           preferred_element_type=_f32)


def _inv_deg(degp):
    deg = jnp.sum(degp.reshape(NW, RB), axis=0)
    return jnp.where(deg > 0, 1.0 / jnp.maximum(deg, 1.0), 0.0)


def _tc1_body(aggp_ref, x_ref, degp_ref, wl_ref, wr_ref, b_ref, h_ref):
    inv = _inv_deg(degp_ref[...])
    mean = (aggp_ref[0] + aggp_ref[1]) * inv[:, None]
    h = _dot(mean, wl_ref[...]) + _dot(x_ref[...], wr_ref[...]) + b_ref[...]
    h_ref[...] = jnp.maximum(h, 0.0)


def _tc1(aggp, x, degp4, wlT, wrT, b):
    return pl.pallas_call(
        _tc1_body,
        grid=(NRB,),
        in_specs=[
            pl.BlockSpec((NC, RB, D), lambda i: (0, i, 0)),
            pl.BlockSpec((RB, D), lambda i: (i, 0)),
            pl.BlockSpec((NW, 1, 1, RB), lambda i: (0, i, 0, 0)),
            pl.BlockSpec((D, D), lambda i: (0, 0)),
            pl.BlockSpec((D, D), lambda i: (0, 0)),
            pl.BlockSpec((1, D), lambda i: (0, 0)),
        ],
        out_specs=pl.BlockSpec((RB, D), lambda i: (i, 0)),
        out_shape=jax.ShapeDtypeStruct((N, D), _f32),
    )(aggp, x, degp4, wlT, wrT, b)


def _one_hot(gi_ref):
    gi = gi_ref[...].reshape(RB)
    return (gi[:, None] == lax.broadcasted_iota(jnp.int32, (RB, G), 1)
            ).astype(_f32)


def _tc2_body(aggp_ref, h_ref, degp_ref, wl_ref, wr_ref, b_ref,
              wa_ref, wr2_ref, gi_ref, a_ref, r_ref, gp_ref):
    i = pl.program_id(0)
    inv = _inv_deg(degp_ref[...])
    mean = (aggp_ref[0] + aggp_ref[1]) * inv[:, None]
    emb = _dot(mean, wl_ref[...]) + _dot(h_ref[...], wr_ref[...]) + b_ref[...]
    a_ref[...] = jnp.sum(emb * wa_ref[...], axis=1).reshape(1, 1, RB)
    r_ref[...] = jnp.sum(emb * wr2_ref[...], axis=1).reshape(1, 1, RB)
    oh = _one_hot(gi_ref)

    @pl.when(i == 0)
    def _():
        gp_ref[...] = jnp.zeros_like(gp_ref)

    gp_ref[...] += lax.dot_general(oh, emb, (((0,), (0,)), ((), ())),
                                   precision=lax.Precision.HIGHEST,
                                   preferred_element_type=_f32)


def _tc2(aggp, h, degp4, wlT, wrT, b, wal, war, gi3):
    return pl.pallas_call(
        _tc2_body,
        grid=(NRB,),
        in_specs=[
            pl.BlockSpec((NC, RB, D), lambda i: (0, i, 0)),
            pl.BlockSpec((RB, D), lambda i: (i, 0)),
            pl.BlockSpec((NW, 1, 1, RB), lambda i: (0, i, 0, 0)),
            pl.BlockSpec((D, D), lambda i: (0, 0)),
            pl.BlockSpec((D, D), lambda i: (0, 0)),
            pl.BlockSpec((1, D), lambda i: (0, 0)),
            pl.BlockSpec((1, D), lambda i: (0, 0)),
            pl.BlockSpec((1, D), lambda i: (0, 0)),
            pl.BlockSpec((1, 1, RB), lambda i: (i, 0, 0)),
        ],
        out_specs=[
            pl.BlockSpec((1, 1, RB), lambda i: (i, 0, 0)),
            pl.BlockSpec((1, 1, RB), lambda i: (i, 0, 0)),
            pl.BlockSpec((G, D), lambda i: (0, 0)),
        ],
        out_shape=[
            jax.ShapeDtypeStruct((NRB, 1, RB), _f32),
            jax.ShapeDtypeStruct((NRB, 1, RB), _f32),
            jax.ShapeDtypeStruct((G, D), _f32),
        ],
    )(aggp, h, degp4, wlT, wrT, b, wal, war, gi3)


def _tc3a_body(advp_ref, degp_ref, r_ref, gp_ref, wv_ref, ba_ref, bv_ref,
               gi_ref, adv_ref, asum_ref, cnt_ref, val_ref):
    i = pl.program_id(0)
    inv = _inv_deg(degp_ref[...])
    agg_a = jnp.sum(advp_ref[...].reshape(NW, RB), axis=0)
    r = r_ref[...].reshape(RB)
    adv = 2.0 * jnp.tanh(agg_a * inv + r + ba_ref[0, 0])
    adv_ref[...] = adv.reshape(1, 1, RB)
    oh = _one_hot(gi_ref)

    @pl.when(i == 0)
    def _():
        asum_ref[...] = jnp.zeros_like(asum_ref)
        cnt_ref[...] = jnp.zeros_like(cnt_ref)
        gp = gp_ref[...]
        val_ref[...] = jnp.tanh(
            jnp.sum(gp * wv_ref[...], axis=1) + bv_ref[0, 0]).reshape(1, G)

    asum_ref[...] += jnp.sum(oh * adv[:, None], axis=0).reshape(1, G)
    cnt_ref[...] += jnp.sum(oh, axis=0).reshape(1, G)


def _tc3a(advp4, degp4, r3, gp, wv, ba, bv, gi3):
    return pl.pallas_call(
        _tc3a_body,
        grid=(NRB,),
        in_specs=[
            pl.BlockSpec((NW, 1, 1, RB), lambda i: (0, i, 0, 0)),
            pl.BlockSpec((NW, 1, 1, RB), lambda i: (0, i, 0, 0)),
            pl.BlockSpec((1, 1, RB), lambda i: (i, 0, 0)),
            pl.BlockSpec((G, D), lambda i: (0, 0)),
            pl.BlockSpec((1, D), lambda i: (0, 0)),
            pl.BlockSpec((1, 1), lambda i: (0, 0)),
            pl.BlockSpec((1, 1), lambda i: (0, 0)),
            pl.BlockSpec((1, 1, RB), lambda i: (i, 0, 0)),
        ],
        out_specs=[
            pl.BlockSpec((1, 1, RB), lambda i: (i, 0, 0)),
            pl.BlockSpec((1, G), lambda i: (0, 0)),
            pl.BlockSpec((1, G), lambda i: (0, 0)),
            pl.BlockSpec((1, G), lambda i: (0, 0)),
        ],
        out_shape=[
            jax.ShapeDtypeStruct((NRB, 1, RB), _f32),
            jax.ShapeDtypeStruct((1, G), _f32),
            jax.ShapeDtypeStruct((1, G), _f32),
            jax.ShapeDtypeStruct((1, G), _f32),
        ],
    )(advp4, degp4, r3, gp, wv, ba, bv, gi3)


def _tc3b_body(adv_ref, val_ref, asum_ref, cnt_ref, gi_ref, out_ref):
    cnt = cnt_ref[...].reshape(G)
    asum = asum_ref[...].reshape(G)
    amean = jnp.where(cnt > 0, asum / jnp.maximum(cnt, 1.0), 0.0)
    corr = val_ref[...].reshape(G) - amean
    oh = _one_hot(gi_ref)
    per_row = jnp.sum(oh * corr[None, :], axis=1)
    out_ref[...] = jnp.tanh(per_row + adv_ref[...].reshape(RB)
                            ).reshape(1, 1, RB)


def _tc3b(adv3, val, asum, cnt, gi3):
    return pl.pallas_call(
        _tc3b_body,
        grid=(NRB,),
        in_specs=[
            pl.BlockSpec((1, 1, RB), lambda i: (i, 0, 0)),
            pl.BlockSpec((1, G), lambda i: (0, 0)),
            pl.BlockSpec((1, G), lambda i: (0, 0)),
            pl.BlockSpec((1, G), lambda i: (0, 0)),
            pl.BlockSpec((1, 1, RB), lambda i: (i, 0, 0)),
        ],
        out_specs=pl.BlockSpec((1, 1, RB), lambda i: (i, 0, 0)),
        out_shape=jax.ShapeDtypeStruct((NRB, 1, RB), _f32),
    )(adv3, val, asum, cnt, gi3)


def kernel(x, edge_index, graph_indices, W1l, W1r, b1, W2l, W2r, b2,
           Wal, War, ba, Wv, bv):
    src = edge_index[0]
    dst = edge_index[1]
    z2d = jnp.zeros((RPT, D), _f32)
    zn = jnp.zeros((N,), _f32)
    gi3 = graph_indices.reshape(NRB, 1, RB)

    agg1p, degp = _sc_seg_with_deg(x, src, dst, z2d, zn)
    degp4 = degp.reshape(NW, NRB, 1, RB)
    h = _tc1(agg1p, x, degp4, W1l.T, W1r.T, b1.reshape(1, D))
    agg2p = _sc_seg(h, src, dst, z2d)
    a3, r3, gp = _tc2(agg2p, h, degp4, W2l.T, W2r.T, b2.reshape(1, D),
                      Wal, War, gi3)
    advp = _sc_adv(a3.reshape(N), src, dst, zn)
    adv3, asum, cnt, val = _tc3a(advp.reshape(NW, NRB, 1, RB), degp4, r3, gp,
                                 Wv, ba.reshape(1, 1), bv.reshape(1, 1), gi3)
    out3 = _tc3b(adv3, val, asum, cnt, gi3)
    return out3.reshape(N)


# speedup vs baseline: 14.9767x; 1.3480x over previous
import functools

import jax
import jax.numpy as jnp
from jax import lax
from jax.experimental import pallas as pl
from jax.experimental.pallas import tpu as pltpu
from jax.experimental.pallas import tpu_sc as plsc

N, E, D, G = 10000, 320000, 128, 64
NC, NS = 2, 16
NW = NC * NS
EPW = E // NW
KC = 80
NCHUNK = EPW // KC
NPAD = 10240
RPT = NPAD // NS
RB = 1000
NRB = N // RB

_f32 = jnp.float32
_mesh = plsc.VectorSubcoreMesh(core_axis_name="c", subcore_axis_name="s")
_sc_params = pltpu.CompilerParams(needs_layout_passes=False)


NBUF = 4
NIDX = 8


def _seg_body(with_deg, x_hbm, src_hbm, dst_hbm, z2d_hbm, zn_hbm, ones_hbm,
              *refs):
    if with_deg:
        (agg_out, deg_out, ones_v) = refs[:3]
        rest = refs[3:]
    else:
        agg_out = refs[0]
        rest = refs[1:]
        ones_v = None
    srcbs = rest[:NIDX]
    dstbs = rest[NIDX:2 * NIDX]
    rowss = rest[2 * NIDX:2 * NIDX + NBUF]
    k = 2 * NIDX + NBUF
    accum = rest[k]
    k += 1
    if with_deg:
        degsh = rest[k]
        k += 1
    else:
        degsh = None
    sgs = rest[k:k + NBUF]
    sds = rest[k + NBUF:k + NBUF + NIDX]
    sss = rest[k + NBUF + NIDX:k + NBUF + 2 * NIDX]
    sa = rest[k + NBUF + 2 * NIDX]
    sb = rest[k + NBUF + 2 * NIDX + 1] if with_deg else None
    c = lax.axis_index("c")
    s = lax.axis_index("s")
    wid = s * NC + c
    base_e = wid * EPW

    pltpu.sync_copy(z2d_hbm, accum.at[pl.ds(s * RPT, RPT)])
    if with_deg:
        pltpu.sync_copy(zn_hbm.at[pl.ds(0, RPT)],
                        degsh.at[pl.ds(s * RPT, RPT)])
        pltpu.sync_copy(ones_hbm, ones_v)
    plsc.subcore_barrier()

    def issue_idx(i, b):
        pltpu.async_copy(src_hbm.at[pl.ds(base_e + i * KC, KC)],
                         srcbs[b], sss[b])
        pltpu.async_copy(dst_hbm.at[pl.ds(base_e + i * KC, KC)],
                         dstbs[b], sds[b])

    def issue_gather(i, b8, b4):
        pltpu.make_async_copy(src_hbm.at[pl.ds(0, KC)], srcbs[b8],
                              sss[b8]).wait()
        pltpu.async_copy(x_hbm.at[srcbs[b8].at[pl.ds(0, KC)]],
                         rowss[b4], sgs[b4])

    def step(i, b8, b4):
        pltpu.make_async_copy(x_hbm.at[pl.ds(0, KC)], rowss[b4],
                              sgs[b4]).wait()
        pltpu.make_async_copy(dst_hbm.at[pl.ds(0, KC)], dstbs[b8],
                              sds[b8]).wait()
        da = pltpu.async_copy(rowss[b4], accum.at[dstbs[b8]], sa, add=True)
        if with_deg:
            db = pltpu.async_copy(ones_v, degsh.at[dstbs[b8]], sb, add=True)
        da.wait()
        if with_deg:
            db.wait()
        ib = (b8 + NIDX - 1) % NIDX
        gb8 = (b8 + NBUF - 1) % NIDX
        gb4 = (b4 + NBUF - 1) % NBUF
        if isinstance(i, int):
            if i + NIDX - 1 < NCHUNK:
                issue_idx(i + NIDX - 1, ib)
            if i + NBUF - 1 < NCHUNK:
                issue_gather(i + NBUF - 1, gb8, gb4)
        else:
            @pl.when(i + NIDX - 1 < NCHUNK)
            def _():
                issue_idx(i + NIDX - 1, ib)

            @pl.when(i + NBUF - 1 < NCHUNK)
            def _():
                issue_gather(i + NBUF - 1, gb8, gb4)

    for i in range(NIDX - 1):
        issue_idx(i, i)
    for i in range(NBUF - 1):
        issue_gather(i, i, i)

    def body(j, carry):
        for b in range(NIDX):
            i = NIDX * j + b
            step(i, b, b % NBUF)
        return carry

    nloop = NCHUNK // NIDX
    lax.fori_loop(0, nloop, body, 0)
    for i in range(nloop * NIDX, NCHUNK):
        step(i, i % NIDX, i % NBUF)
    plsc.subcore_barrier()

    pltpu.sync_copy(accum.at[pl.ds(s * RPT, RPT)],
                    agg_out.at[c, pl.ds(s * RPT, RPT)])
    if with_deg:
        pltpu.sync_copy(degsh.at[pl.ds(s * RPT, RPT)],
                        deg_out.at[c, pl.ds(s * RPT, RPT)])


_SEG_SEMS = [pltpu.SemaphoreType.DMA] * (NBUF + 2 * NIDX + 1)


def _sc_seg_with_deg(x, src, dst, z2d, zn, ones):
    out_type = (jax.ShapeDtypeStruct((NC, NPAD, D), _f32),
                jax.ShapeDtypeStruct((NC, NPAD), _f32))
    scratch = ([pltpu.VMEM((KC,), _f32)]
               + [pltpu.VMEM((KC,), jnp.int32)] * (2 * NIDX)
               + [pltpu.VMEM((KC, D), _f32)] * NBUF
               + [pltpu.VMEM_SHARED((NPAD, D), _f32),
                  pltpu.VMEM_SHARED((NPAD,), _f32)]
               + _SEG_SEMS + [pltpu.SemaphoreType.DMA])
    f = pl.kernel(functools.partial(_seg_body, True), out_type=out_type,
                  mesh=_mesh, scratch_types=scratch, compiler_params=_sc_params)
    return f(x, src, dst, z2d, zn, ones)


def _seg_body_nodeg(x_hbm, src_hbm, dst_hbm, z2d_hbm, *refs):
    _seg_body(False, x_hbm, src_hbm, dst_hbm, z2d_hbm, None, None, *refs)


def _sc_seg(x, src, dst, z2d):
    out_type = jax.ShapeDtypeStruct((NC, NPAD, D), _f32)
    scratch = ([pltpu.VMEM((KC,), jnp.int32)] * (2 * NIDX)
               + [pltpu.VMEM((KC, D), _f32)] * NBUF
               + [pltpu.VMEM_SHARED((NPAD, D), _f32)]
               + _SEG_SEMS)
    f = pl.kernel(_seg_body_nodeg, out_type=out_type,
                  mesh=_mesh, scratch_types=scratch, compiler_params=_sc_params)
    return f(x, src, dst, z2d)


_NG = EPW // 16
_ADV_UNROLL = 5


def _adv_body(a_hbm, src_hbm, dst_hbm, zn_hbm, out_hbm,
              a_v, accum_v, src1_v, dst1_v):
    c = lax.axis_index("c")
    s = lax.axis_index("s")
    wid = s * NC + c
    pltpu.sync_copy(a_hbm, a_v)
    pltpu.sync_copy(zn_hbm, accum_v)
    pltpu.sync_copy(src_hbm.at[pl.ds(wid * EPW, EPW)], src1_v)
    pltpu.sync_copy(dst_hbm.at[pl.ds(wid * EPW, EPW)], dst1_v)

    def group(j, carry):
        for v in range(_ADV_UNROLL):
            off = (j * _ADV_UNROLL + v) * 16
            vals = plsc.load_gather(a_v, [src1_v[pl.ds(off, 16)]])
            plsc.addupdate_scatter(accum_v, [dst1_v[pl.ds(off, 16)]], vals)
        return carry

    lax.fori_loop(0, _NG // _ADV_UNROLL, group, 0)
    pltpu.sync_copy(accum_v, out_hbm.at[wid])


def _sc_adv(a, src, dst, zn):
    out_type = jax.ShapeDtypeStruct((NW, N), _f32)
    scratch = [
        pltpu.VMEM((N,), _f32),
        pltpu.VMEM((N,), _f32),
        pltpu.VMEM((EPW,), jnp.int32),
        pltpu.VMEM((EPW,), jnp.int32),
    ]
    f = pl.kernel(_adv_body, out_type=out_type, mesh=_mesh,
                  scratch_types=scratch, compiler_params=_sc_params)
    return f(a, src, dst, zn)


def _dot(a, b):
    return lax.dot_general(a, b, (((1,), (0,)), ((), ())),
                           precision=lax.Precision.HIGHEST,
                           preferred_element_type=_f32)


def _inv_deg(degp):
    deg = jnp.sum(degp.reshape(NC, RB), axis=0)
    return jnp.where(deg > 0, 1.0 / jnp.maximum(deg, 1.0), 0.0)


def _tc1_body(aggp_ref, x_ref, degp_ref, wl_ref, wr_ref, b_ref, h_ref):
    inv = _inv_deg(degp_ref[...])
    mean = (aggp_ref[0] + aggp_ref[1]) * inv[:, None]
    h = _dot(mean, wl_ref[...]) + _dot(x_ref[...], wr_ref[...]) + b_ref[...]
    h_ref[...] = jnp.maximum(h, 0.0)


def _tc1(aggp, x, degp4, wlT, wrT, b):
    return pl.pallas_call(
        _tc1_body,
        grid=(NRB,),
        in_specs=[
            pl.BlockSpec((NC, RB, D), lambda i: (0, i, 0)),
            pl.BlockSpec((RB, D), lambda i: (i, 0)),
            pl.BlockSpec((NC, 1, 1, RB), lambda i: (0, i, 0, 0)),
            pl.BlockSpec((D, D), lambda i: (0, 0)),
            pl.BlockSpec((D, D), lambda i: (0, 0)),
            pl.BlockSpec((1, D), lambda i: (0, 0)),
        ],
        out_specs=pl.BlockSpec((RB, D), lambda i: (i, 0)),
        out_shape=jax.ShapeDtypeStruct((N, D), _f32),
    )(aggp, x, degp4, wlT, wrT, b)


def _one_hot(gi_ref):
    gi = gi_ref[...].reshape(RB)
    return (gi[:, None] == lax.broadcasted_iota(jnp.int32, (RB, G), 1)
            ).astype(_f32)


def _tc2_body(aggp_ref, h_ref, degp_ref, wl_ref, wr_ref, b_ref,
              wa_ref, wr2_ref, gi_ref, a_ref, r_ref, gp_ref):
    i = pl.program_id(0)
    inv = _inv_deg(degp_ref[...])
    mean = (aggp_ref[0] + aggp_ref[1]) * inv[:, None]
    emb = _dot(mean, wl_ref[...]) + _dot(h_ref[...], wr_ref[...]) + b_ref[...]
    a_ref[...] = jnp.sum(emb * wa_ref[...], axis=1).reshape(1, 1, RB)
    r_ref[...] = jnp.sum(emb * wr2_ref[...], axis=1).reshape(1, 1, RB)
    oh = _one_hot(gi_ref)

    @pl.when(i == 0)
    def _():
        gp_ref[...] = jnp.zeros_like(gp_ref)

    gp_ref[...] += lax.dot_general(oh, emb, (((0,), (0,)), ((), ())),
                                   precision=lax.Precision.HIGHEST,
                                   preferred_element_type=_f32)


def _tc2(aggp, h, degp4, wlT, wrT, b, wal, war, gi3):
    return pl.pallas_call(
        _tc2_body,
        grid=(NRB,),
        in_specs=[
            pl.BlockSpec((NC, RB, D), lambda i: (0, i, 0)),
            pl.BlockSpec((RB, D), lambda i: (i, 0)),
            pl.BlockSpec((NC, 1, 1, RB), lambda i: (0, i, 0, 0)),
            pl.BlockSpec((D, D), lambda i: (0, 0)),
            pl.BlockSpec((D, D), lambda i: (0, 0)),
            pl.BlockSpec((1, D), lambda i: (0, 0)),
            pl.BlockSpec((1, D), lambda i: (0, 0)),
            pl.BlockSpec((1, D), lambda i: (0, 0)),
            pl.BlockSpec((1, 1, RB), lambda i: (i, 0, 0)),
        ],
        out_specs=[
            pl.BlockSpec((1, 1, RB), lambda i: (i, 0, 0)),
            pl.BlockSpec((1, 1, RB), lambda i: (i, 0, 0)),
            pl.BlockSpec((G, D), lambda i: (0, 0)),
        ],
        out_shape=[
            jax.ShapeDtypeStruct((NRB, 1, RB), _f32),
            jax.ShapeDtypeStruct((NRB, 1, RB), _f32),
            jax.ShapeDtypeStruct((G, D), _f32),
        ],
    )(aggp, h, degp4, wlT, wrT, b, wal, war, gi3)


def _tc3a_body(advp_ref, degp_ref, r_ref, gp_ref, wv_ref, ba_ref, bv_ref,
               gi_ref, adv_ref, asum_ref, cnt_ref, val_ref):
    i = pl.program_id(0)
    inv = _inv_deg(degp_ref[...])
    agg_a = jnp.sum(advp_ref[...].reshape(NW, RB), axis=0)
    r = r_ref[...].reshape(RB)
    adv = 2.0 * jnp.tanh(agg_a * inv + r + ba_ref[0, 0])
    adv_ref[...] = adv.reshape(1, 1, RB)
    oh = _one_hot(gi_ref)

    @pl.when(i == 0)
    def _():
        asum_ref[...] = jnp.zeros_like(asum_ref)
        cnt_ref[...] = jnp.zeros_like(cnt_ref)
        gp = gp_ref[...]
        val_ref[...] = jnp.tanh(
            jnp.sum(gp * wv_ref[...], axis=1) + bv_ref[0, 0]).reshape(1, G)

    asum_ref[...] += jnp.sum(oh * adv[:, None], axis=0).reshape(1, G)
    cnt_ref[...] += jnp.sum(oh, axis=0).reshape(1, G)


def _tc3a(advp4, degp4, r3, gp, wv, ba, bv, gi3):
    return pl.pallas_call(
        _tc3a_body,
        grid=(NRB,),
        in_specs=[
            pl.BlockSpec((NW, 1, 1, RB), lambda i: (0, i, 0, 0)),
            pl.BlockSpec((NC, 1, 1, RB), lambda i: (0, i, 0, 0)),
            pl.BlockSpec((1, 1, RB), lambda i: (i, 0, 0)),
            pl.BlockSpec((G, D), lambda i: (0, 0)),
            pl.BlockSpec((1, D), lambda i: (0, 0)),
            pl.BlockSpec((1, 1), lambda i: (0, 0)),
            pl.BlockSpec((1, 1), lambda i: (0, 0)),
            pl.BlockSpec((1, 1, RB), lambda i: (i, 0, 0)),
        ],
        out_specs=[
            pl.BlockSpec((1, 1, RB), lambda i: (i, 0, 0)),
            pl.BlockSpec((1, G), lambda i: (0, 0)),
            pl.BlockSpec((1, G), lambda i: (0, 0)),
            pl.BlockSpec((1, G), lambda i: (0, 0)),
        ],
        out_shape=[
            jax.ShapeDtypeStruct((NRB, 1, RB), _f32),
            jax.ShapeDtypeStruct((1, G), _f32),
            jax.ShapeDtypeStruct((1, G), _f32),
            jax.ShapeDtypeStruct((1, G), _f32),
        ],
    )(advp4, degp4, r3, gp, wv, ba, bv, gi3)


def _tc3b_body(adv_ref, val_ref, asum_ref, cnt_ref, gi_ref, out_ref):
    cnt = cnt_ref[...].reshape(G)
    asum = asum_ref[...].reshape(G)
    amean = jnp.where(cnt > 0, asum / jnp.maximum(cnt, 1.0), 0.0)
    corr = val_ref[...].reshape(G) - amean
    oh = _one_hot(gi_ref)
    per_row = jnp.sum(oh * corr[None, :], axis=1)
    out_ref[...] = jnp.tanh(per_row + adv_ref[...].reshape(RB)
                            ).reshape(1, 1, RB)


def _tc3b(adv3, val, asum, cnt, gi3):
    return pl.pallas_call(
        _tc3b_body,
        grid=(NRB,),
        in_specs=[
            pl.BlockSpec((1, 1, RB), lambda i: (i, 0, 0)),
            pl.BlockSpec((1, G), lambda i: (0, 0)),
            pl.BlockSpec((1, G), lambda i: (0, 0)),
            pl.BlockSpec((1, G), lambda i: (0, 0)),
            pl.BlockSpec((1, 1, RB), lambda i: (i, 0, 0)),
        ],
        out_specs=pl.BlockSpec((1, 1, RB), lambda i: (i, 0, 0)),
        out_shape=jax.ShapeDtypeStruct((NRB, 1, RB), _f32),
    )(adv3, val, asum, cnt, gi3)


def kernel(x, edge_index, graph_indices, W1l, W1r, b1, W2l, W2r, b2,
           Wal, War, ba, Wv, bv):
    src = edge_index[0]
    dst = edge_index[1]
    z2d = jnp.zeros((RPT, D), _f32)
    zn = jnp.zeros((N,), _f32)
    ones = jnp.ones((KC,), _f32)
    gi3 = graph_indices.reshape(NRB, 1, RB)

    agg1p, degp = _sc_seg_with_deg(x, src, dst, z2d, zn, ones)
    degp4 = degp[:, :N].reshape(NC, NRB, 1, RB)
    h = _tc1(agg1p, x, degp4, W1l.T, W1r.T, b1.reshape(1, D))
    agg2p = _sc_seg(h, src, dst, z2d)
    a3, r3, gp = _tc2(agg2p, h, degp4, W2l.T, W2r.T, b2.reshape(1, D),
                      Wal, War, gi3)
    advp = _sc_adv(a3.reshape(N), src, dst, zn)
    adv3, asum, cnt, val = _tc3a(advp.reshape(NW, NRB, 1, RB), degp4, r3, gp,
                                 Wv, ba.reshape(1, 1), bv.reshape(1, 1), gi3)
    out3 = _tc3b(adv3, val, asum, cnt, gi3)
    return out3.reshape(N)


# speedup vs baseline: 15.2902x; 1.0209x over previous
import functools

import jax
import jax.numpy as jnp
from jax import lax
from jax.experimental import pallas as pl
from jax.experimental.pallas import tpu as pltpu
from jax.experimental.pallas import tpu_sc as plsc

N, E, D, G = 10000, 320000, 128, 64
NC, NS = 2, 16
NW = NC * NS
EPW = E // NW
KC = 80
NCHUNK = EPW // KC
NPAD = 10240
RPT = NPAD // NS
RB = 1000
NRB = N // RB

_f32 = jnp.float32
_mesh = plsc.VectorSubcoreMesh(core_axis_name="c", subcore_axis_name="s")
_sc_params = pltpu.CompilerParams(needs_layout_passes=False)


NBUF = 4
NIDX = 8


def _seg_body(with_deg, x_hbm, src_hbm, dst_hbm, z2d_hbm, zn_hbm, ones_hbm,
              *refs):
    if with_deg:
        (agg_out, deg_out, ones_v) = refs[:3]
        rest = refs[3:]
    else:
        agg_out = refs[0]
        rest = refs[1:]
        ones_v = None
    srcbs = rest[:NIDX]
    dstbs = rest[NIDX:2 * NIDX]
    rowss = rest[2 * NIDX:2 * NIDX + NBUF]
    k = 2 * NIDX + NBUF
    accum = rest[k]
    k += 1
    if with_deg:
        degsh = rest[k]
        k += 1
    else:
        degsh = None
    sgs = rest[k:k + NBUF]
    sds = rest[k + NBUF:k + NBUF + NIDX]
    sss = rest[k + NBUF + NIDX:k + NBUF + 2 * NIDX]
    sas = rest[k + NBUF + 2 * NIDX:k + NBUF + 2 * NIDX + 2]
    sbs = rest[k + NBUF + 2 * NIDX + 2:k + NBUF + 2 * NIDX + 4] \
        if with_deg else None
    c = lax.axis_index("c")
    s = lax.axis_index("s")
    wid = s * NC + c
    base_e = wid * EPW

    pltpu.sync_copy(z2d_hbm, accum.at[pl.ds(s * RPT, RPT)])
    if with_deg:
        pltpu.sync_copy(zn_hbm.at[pl.ds(0, RPT)],
                        degsh.at[pl.ds(s * RPT, RPT)])
        pltpu.sync_copy(ones_hbm, ones_v)
    plsc.subcore_barrier()

    def issue_idx(i, b):
        pltpu.async_copy(src_hbm.at[pl.ds(base_e + i * KC, KC)],
                         srcbs[b], sss[b])
        pltpu.async_copy(dst_hbm.at[pl.ds(base_e + i * KC, KC)],
                         dstbs[b], sds[b])

    def issue_gather(i, b8, b4):
        pltpu.make_async_copy(src_hbm.at[pl.ds(0, KC)], srcbs[b8],
                              sss[b8]).wait()
        pltpu.async_copy(x_hbm.at[srcbs[b8].at[pl.ds(0, KC)]],
                         rowss[b4], sgs[b4])

    def wait_pending(pending):
        if pending is not None:
            for d in pending:
                d.wait()

    def step(i, b8, b4, pending):
        pltpu.make_async_copy(x_hbm.at[pl.ds(0, KC)], rowss[b4],
                              sgs[b4]).wait()
        pltpu.make_async_copy(dst_hbm.at[pl.ds(0, KC)], dstbs[b8],
                              sds[b8]).wait()
        wait_pending(pending)
        nxt = [pltpu.async_copy(rowss[b4], accum.at[dstbs[b8]],
                                sas[b8 % 2], add=True)]
        if with_deg:
            nxt.append(pltpu.async_copy(ones_v, degsh.at[dstbs[b8]],
                                        sbs[b8 % 2], add=True))
        ib = (b8 + NIDX - 1) % NIDX
        gb8 = (b8 + NBUF - 1) % NIDX
        gb4 = (b4 + NBUF - 1) % NBUF
        if isinstance(i, int):
            if i + NIDX - 1 < NCHUNK:
                issue_idx(i + NIDX - 1, ib)
            if i + NBUF - 1 < NCHUNK:
                issue_gather(i + NBUF - 1, gb8, gb4)
        else:
            @pl.when(i + NIDX - 1 < NCHUNK)
            def _():
                issue_idx(i + NIDX - 1, ib)

            @pl.when(i + NBUF - 1 < NCHUNK)
            def _():
                issue_gather(i + NBUF - 1, gb8, gb4)
        return nxt

    for i in range(NIDX - 1):
        issue_idx(i, i)
    for i in range(NBUF - 1):
        issue_gather(i, i, i)

    def body(j, carry):
        pending = None
        for b in range(NIDX):
            i = NIDX * j + b
            pending = step(i, b, b % NBUF, pending)
        wait_pending(pending)
        return carry

    nloop = NCHUNK // NIDX
    lax.fori_loop(0, nloop, body, 0)
    pending = None
    for i in range(nloop * NIDX, NCHUNK):
        pending = step(i, i % NIDX, i % NBUF, pending)
    wait_pending(pending)
    plsc.subcore_barrier()

    pltpu.sync_copy(accum.at[pl.ds(s * RPT, RPT)],
                    agg_out.at[c, pl.ds(s * RPT, RPT)])
    if with_deg:
        pltpu.sync_copy(degsh.at[pl.ds(s * RPT, RPT)],
                        deg_out.at[c, pl.ds(s * RPT, RPT)])


_SEG_SEMS = [pltpu.SemaphoreType.DMA] * (NBUF + 2 * NIDX + 2)


def _sc_seg_with_deg(x, src, dst, z2d, zn, ones):
    out_type = (jax.ShapeDtypeStruct((NC, NPAD, D), _f32),
                jax.ShapeDtypeStruct((NC, NPAD), _f32))
    scratch = ([pltpu.VMEM((KC,), _f32)]
               + [pltpu.VMEM((KC,), jnp.int32)] * (2 * NIDX)
               + [pltpu.VMEM((KC, D), _f32)] * NBUF
               + [pltpu.VMEM_SHARED((NPAD, D), _f32),
                  pltpu.VMEM_SHARED((NPAD,), _f32)]
               + _SEG_SEMS + [pltpu.SemaphoreType.DMA] * 2)
    f = pl.kernel(functools.partial(_seg_body, True), out_type=out_type,
                  mesh=_mesh, scratch_types=scratch, compiler_params=_sc_params)
    return f(x, src, dst, z2d, zn, ones)


def _seg_body_nodeg(x_hbm, src_hbm, dst_hbm, z2d_hbm, *refs):
    _seg_body(False, x_hbm, src_hbm, dst_hbm, z2d_hbm, None, None, *refs)


def _sc_seg(x, src, dst, z2d):
    out_type = jax.ShapeDtypeStruct((NC, NPAD, D), _f32)
    scratch = ([pltpu.VMEM((KC,), jnp.int32)] * (2 * NIDX)
               + [pltpu.VMEM((KC, D), _f32)] * NBUF
               + [pltpu.VMEM_SHARED((NPAD, D), _f32)]
               + _SEG_SEMS)
    f = pl.kernel(_seg_body_nodeg, out_type=out_type,
                  mesh=_mesh, scratch_types=scratch, compiler_params=_sc_params)
    return f(x, src, dst, z2d)


_NG = EPW // 16
_ADV_UNROLL = 5


def _adv_body(a_hbm, src_hbm, dst_hbm, zn_hbm, out_hbm,
              a_v, accum_v, src1_v, dst1_v):
    c = lax.axis_index("c")
    s = lax.axis_index("s")
    wid = s * NC + c
    pltpu.sync_copy(a_hbm, a_v)
    pltpu.sync_copy(zn_hbm, accum_v)
    pltpu.sync_copy(src_hbm.at[pl.ds(wid * EPW, EPW)], src1_v)
    pltpu.sync_copy(dst_hbm.at[pl.ds(wid * EPW, EPW)], dst1_v)

    def group(j, carry):
        for v in range(_ADV_UNROLL):
            off = (j * _ADV_UNROLL + v) * 16
            vals = plsc.load_gather(a_v, [src1_v[pl.ds(off, 16)]])
            plsc.addupdate_scatter(accum_v, [dst1_v[pl.ds(off, 16)]], vals)
        return carry

    lax.fori_loop(0, _NG // _ADV_UNROLL, group, 0)
    pltpu.sync_copy(accum_v, out_hbm.at[wid])


def _sc_adv(a, src, dst, zn):
    out_type = jax.ShapeDtypeStruct((NW, N), _f32)
    scratch = [
        pltpu.VMEM((N,), _f32),
        pltpu.VMEM((N,), _f32),
        pltpu.VMEM((EPW,), jnp.int32),
        pltpu.VMEM((EPW,), jnp.int32),
    ]
    f = pl.kernel(_adv_body, out_type=out_type, mesh=_mesh,
                  scratch_types=scratch, compiler_params=_sc_params)
    return f(a, src, dst, zn)


def _dot(a, b):
    return lax.dot_general(a, b, (((1,), (0,)), ((), ())),
                           precision=lax.Precision.HIGHEST,
                           preferred_element_type=_f32)


def _inv_deg(degp):
    deg = jnp.sum(degp.reshape(NC, RB), axis=0)
    return jnp.where(deg > 0, 1.0 / jnp.maximum(deg, 1.0), 0.0)


def _tc1_body(aggp_ref, x_ref, degp_ref, wl_ref, wr_ref, b_ref, h_ref):
    inv = _inv_deg(degp_ref[...])
    mean = (aggp_ref[0] + aggp_ref[1]) * inv[:, None]
    h = _dot(mean, wl_ref[...]) + _dot(x_ref[...], wr_ref[...]) + b_ref[...]
    h_ref[...] = jnp.maximum(h, 0.0)


def _tc1(aggp, x, degp4, wlT, wrT, b):
    return pl.pallas_call(
        _tc1_body,
        grid=(NRB,),
        in_specs=[
            pl.BlockSpec((NC, RB, D), lambda i: (0, i, 0)),
            pl.BlockSpec((RB, D), lambda i: (i, 0)),
            pl.BlockSpec((NC, 1, 1, RB), lambda i: (0, i, 0, 0)),
            pl.BlockSpec((D, D), lambda i: (0, 0)),
            pl.BlockSpec((D, D), lambda i: (0, 0)),
            pl.BlockSpec((1, D), lambda i: (0, 0)),
        ],
        out_specs=pl.BlockSpec((RB, D), lambda i: (i, 0)),
        out_shape=jax.ShapeDtypeStruct((N, D), _f32),
    )(aggp, x, degp4, wlT, wrT, b)


def _one_hot(gi_ref):
    gi = gi_ref[...].reshape(RB)
    return (gi[:, None] == lax.broadcasted_iota(jnp.int32, (RB, G), 1)
            ).astype(_f32)


def _tc2_body(aggp_ref, h_ref, degp_ref, wl_ref, wr_ref, b_ref,
              wa_ref, wr2_ref, gi_ref, a_ref, r_ref, gp_ref):
    i = pl.program_id(0)
    inv = _inv_deg(degp_ref[...])
    mean = (aggp_ref[0] + aggp_ref[1]) * inv[:, None]
    emb = _dot(mean, wl_ref[...]) + _dot(h_ref[...], wr_ref[...]) + b_ref[...]
    a_ref[...] = jnp.sum(emb * wa_ref[...], axis=1).reshape(1, 1, RB)
    r_ref[...] = jnp.sum(emb * wr2_ref[...], axis=1).reshape(1, 1, RB)
    oh = _one_hot(gi_ref)

    @pl.when(i == 0)
    def _():
        gp_ref[...] = jnp.zeros_like(gp_ref)

    gp_ref[...] += lax.dot_general(oh, emb, (((0,), (0,)), ((), ())),
                                   precision=lax.Precision.HIGHEST,
                                   preferred_element_type=_f32)


def _tc2(aggp, h, degp4, wlT, wrT, b, wal, war, gi3):
    return pl.pallas_call(
        _tc2_body,
        grid=(NRB,),
        in_specs=[
            pl.BlockSpec((NC, RB, D), lambda i: (0, i, 0)),
            pl.BlockSpec((RB, D), lambda i: (i, 0)),
            pl.BlockSpec((NC, 1, 1, RB), lambda i: (0, i, 0, 0)),
            pl.BlockSpec((D, D), lambda i: (0, 0)),
            pl.BlockSpec((D, D), lambda i: (0, 0)),
            pl.BlockSpec((1, D), lambda i: (0, 0)),
            pl.BlockSpec((1, D), lambda i: (0, 0)),
            pl.BlockSpec((1, D), lambda i: (0, 0)),
            pl.BlockSpec((1, 1, RB), lambda i: (i, 0, 0)),
        ],
        out_specs=[
            pl.BlockSpec((1, 1, RB), lambda i: (i, 0, 0)),
            pl.BlockSpec((1, 1, RB), lambda i: (i, 0, 0)),
            pl.BlockSpec((G, D), lambda i: (0, 0)),
        ],
        out_shape=[
            jax.ShapeDtypeStruct((NRB, 1, RB), _f32),
            jax.ShapeDtypeStruct((NRB, 1, RB), _f32),
            jax.ShapeDtypeStruct((G, D), _f32),
        ],
    )(aggp, h, degp4, wlT, wrT, b, wal, war, gi3)


def _tc3a_body(advp_ref, degp_ref, r_ref, gp_ref, wv_ref, ba_ref, bv_ref,
               gi_ref, adv_ref, asum_ref, cnt_ref, val_ref):
    i = pl.program_id(0)
    inv = _inv_deg(degp_ref[...])
    agg_a = jnp.sum(advp_ref[...].reshape(NW, RB), axis=0)
    r = r_ref[...].reshape(RB)
    adv = 2.0 * jnp.tanh(agg_a * inv + r + ba_ref[0, 0])
    adv_ref[...] = adv.reshape(1, 1, RB)
    oh = _one_hot(gi_ref)

    @pl.when(i == 0)
    def _():
        asum_ref[...] = jnp.zeros_like(asum_ref)
        cnt_ref[...] = jnp.zeros_like(cnt_ref)
        gp = gp_ref[...]
        val_ref[...] = jnp.tanh(
            jnp.sum(gp * wv_ref[...], axis=1) + bv_ref[0, 0]).reshape(1, G)

    asum_ref[...] += jnp.sum(oh * adv[:, None], axis=0).reshape(1, G)
    cnt_ref[...] += jnp.sum(oh, axis=0).reshape(1, G)


def _tc3a(advp4, degp4, r3, gp, wv, ba, bv, gi3):
    return pl.pallas_call(
        _tc3a_body,
        grid=(NRB,),
        in_specs=[
            pl.BlockSpec((NW, 1, 1, RB), lambda i: (0, i, 0, 0)),
            pl.BlockSpec((NC, 1, 1, RB), lambda i: (0, i, 0, 0)),
            pl.BlockSpec((1, 1, RB), lambda i: (i, 0, 0)),
            pl.BlockSpec((G, D), lambda i: (0, 0)),
            pl.BlockSpec((1, D), lambda i: (0, 0)),
            pl.BlockSpec((1, 1), lambda i: (0, 0)),
            pl.BlockSpec((1, 1), lambda i: (0, 0)),
            pl.BlockSpec((1, 1, RB), lambda i: (i, 0, 0)),
        ],
        out_specs=[
            pl.BlockSpec((1, 1, RB), lambda i: (i, 0, 0)),
            pl.BlockSpec((1, G), lambda i: (0, 0)),
            pl.BlockSpec((1, G), lambda i: (0, 0)),
            pl.BlockSpec((1, G), lambda i: (0, 0)),
        ],
        out_shape=[
            jax.ShapeDtypeStruct((NRB, 1, RB), _f32),
            jax.ShapeDtypeStruct((1, G), _f32),
            jax.ShapeDtypeStruct((1, G), _f32),
            jax.ShapeDtypeStruct((1, G), _f32),
        ],
    )(advp4, degp4, r3, gp, wv, ba, bv, gi3)


def _tc3b_body(adv_ref, val_ref, asum_ref, cnt_ref, gi_ref, out_ref):
    cnt = cnt_ref[...].reshape(G)
    asum = asum_ref[...].reshape(G)
    amean = jnp.where(cnt > 0, asum / jnp.maximum(cnt, 1.0), 0.0)
    corr = val_ref[...].reshape(G) - amean
    oh = _one_hot(gi_ref)
    per_row = jnp.sum(oh * corr[None, :], axis=1)
    out_ref[...] = jnp.tanh(per_row + adv_ref[...].reshape(RB)
                            ).reshape(1, 1, RB)


def _tc3b(adv3, val, asum, cnt, gi3):
    return pl.pallas_call(
        _tc3b_body,
        grid=(NRB,),
        in_specs=[
            pl.BlockSpec((1, 1, RB), lambda i: (i, 0, 0)),
            pl.BlockSpec((1, G), lambda i: (0, 0)),
            pl.BlockSpec((1, G), lambda i: (0, 0)),
            pl.BlockSpec((1, G), lambda i: (0, 0)),
            pl.BlockSpec((1, 1, RB), lambda i: (i, 0, 0)),
        ],
        out_specs=pl.BlockSpec((1, 1, RB), lambda i: (i, 0, 0)),
        out_shape=jax.ShapeDtypeStruct((NRB, 1, RB), _f32),
    )(adv3, val, asum, cnt, gi3)


def kernel(x, edge_index, graph_indices, W1l, W1r, b1, W2l, W2r, b2,
           Wal, War, ba, Wv, bv):
    src = edge_index[0]
    dst = edge_index[1]
    z2d = jnp.zeros((RPT, D), _f32)
    zn = jnp.zeros((N,), _f32)
    ones = jnp.ones((KC,), _f32)
    gi3 = graph_indices.reshape(NRB, 1, RB)

    agg1p, degp = _sc_seg_with_deg(x, src, dst, z2d, zn, ones)
    degp4 = degp[:, :N].reshape(NC, NRB, 1, RB)
    h = _tc1(agg1p, x, degp4, W1l.T, W1r.T, b1.reshape(1, D))
    agg2p = _sc_seg(h, src, dst, z2d)
    a3, r3, gp = _tc2(agg2p, h, degp4, W2l.T, W2r.T, b2.reshape(1, D),
                      Wal, War, gi3)
    advp = _sc_adv(a3.reshape(N), src, dst, zn)
    adv3, asum, cnt, val = _tc3a(advp.reshape(NW, NRB, 1, RB), degp4, r3, gp,
                                 Wv, ba.reshape(1, 1), bv.reshape(1, 1), gi3)
    out3 = _tc3b(adv3, val, asum, cnt, gi3)
    return out3.reshape(N)


# speedup vs baseline: 15.4261x; 1.0089x over previous
import functools

import jax
import jax.numpy as jnp
from jax import lax
from jax.experimental import pallas as pl
from jax.experimental.pallas import tpu as pltpu
from jax.experimental.pallas import tpu_sc as plsc

N, E, D, G = 10000, 320000, 128, 64
NC, NS = 2, 16
NW = NC * NS
EPW = E // NW
KC = 80
NCHUNK = EPW // KC
NPAD = 10240
RPT = NPAD // NS
RB = 1000
NRB = N // RB

_f32 = jnp.float32
_mesh = plsc.VectorSubcoreMesh(core_axis_name="c", subcore_axis_name="s")
_sc_params = pltpu.CompilerParams(needs_layout_passes=False)


NBUF = 4
NIDX = 8


def _seg_body(with_deg, x_hbm, src_hbm, dst_hbm, z2d_hbm, zn_hbm, ones_hbm,
              *refs):
    if with_deg:
        (agg_out, deg_out, ones_v) = refs[:3]
        rest = refs[3:]
    else:
        agg_out = refs[0]
        rest = refs[1:]
        ones_v = None
    srcbs = rest[:NIDX]
    dstbs = rest[NIDX:2 * NIDX]
    rowss = rest[2 * NIDX:2 * NIDX + NBUF]
    k = 2 * NIDX + NBUF
    accum = rest[k]
    k += 1
    if with_deg:
        degsh = rest[k]
        k += 1
    else:
        degsh = None
    sgs = rest[k:k + NBUF]
    sds = rest[k + NBUF:k + NBUF + NIDX]
    sss = rest[k + NBUF + NIDX:k + NBUF + 2 * NIDX]
    sas = rest[k + NBUF + 2 * NIDX:k + NBUF + 2 * NIDX + 2]
    sbs = rest[k + NBUF + 2 * NIDX + 2:k + NBUF + 2 * NIDX + 4] \
        if with_deg else None
    c = lax.axis_index("c")
    s = lax.axis_index("s")
    wid = s * NC + c
    base_e = wid * EPW

    pltpu.sync_copy(z2d_hbm, accum.at[pl.ds(s * RPT, RPT)])
    if with_deg:
        pltpu.sync_copy(zn_hbm.at[pl.ds(0, RPT)],
                        degsh.at[pl.ds(s * RPT, RPT)])
        pltpu.sync_copy(ones_hbm, ones_v)
    plsc.subcore_barrier()

    def issue_idx(i, b):
        pltpu.async_copy(src_hbm.at[pl.ds(base_e + i * KC, KC)],
                         srcbs[b], sss[b])
        pltpu.async_copy(dst_hbm.at[pl.ds(base_e + i * KC, KC)],
                         dstbs[b], sds[b])

    def issue_gather(i, b8, b4):
        pltpu.make_async_copy(src_hbm.at[pl.ds(0, KC)], srcbs[b8],
                              sss[b8]).wait()
        pltpu.async_copy(x_hbm.at[srcbs[b8].at[pl.ds(0, KC)]],
                         rowss[b4], sgs[b4])

    def wait_pending(pending):
        if pending is not None:
            for d in pending:
                d.wait()

    def step(i, b8, b4, pending):
        pltpu.make_async_copy(x_hbm.at[pl.ds(0, KC)], rowss[b4],
                              sgs[b4]).wait()
        pltpu.make_async_copy(dst_hbm.at[pl.ds(0, KC)], dstbs[b8],
                              sds[b8]).wait()
        wait_pending(pending)
        nxt = [pltpu.async_copy(rowss[b4], accum.at[dstbs[b8]],
                                sas[b8 % 2], add=True)]
        if with_deg:
            nxt.append(pltpu.async_copy(ones_v, degsh.at[dstbs[b8]],
                                        sbs[b8 % 2], add=True))
        ib = (b8 + NIDX - 1) % NIDX
        gb8 = (b8 + NBUF - 1) % NIDX
        gb4 = (b4 + NBUF - 1) % NBUF
        if isinstance(i, int):
            if i + NIDX - 1 < NCHUNK:
                issue_idx(i + NIDX - 1, ib)
            if i + NBUF - 1 < NCHUNK:
                issue_gather(i + NBUF - 1, gb8, gb4)
        else:
            @pl.when(i + NIDX - 1 < NCHUNK)
            def _():
                issue_idx(i + NIDX - 1, ib)

            @pl.when(i + NBUF - 1 < NCHUNK)
            def _():
                issue_gather(i + NBUF - 1, gb8, gb4)
        return nxt

    for i in range(NIDX - 1):
        issue_idx(i, i)
    for i in range(NBUF - 1):
        issue_gather(i, i, i)

    def body(j, carry):
        pending = None
        for b in range(NIDX):
            i = NIDX * j + b
            pending = step(i, b, b % NBUF, pending)
        wait_pending(pending)
        return carry

    nloop = NCHUNK // NIDX
    lax.fori_loop(0, nloop, body, 0)
    pending = None
    for i in range(nloop * NIDX, NCHUNK):
        pending = step(i, i % NIDX, i % NBUF, pending)
    wait_pending(pending)
    plsc.subcore_barrier()

    pltpu.sync_copy(accum.at[pl.ds(s * RPT, RPT)],
                    agg_out.at[c, pl.ds(s * RPT, RPT)])
    if with_deg:
        pltpu.sync_copy(degsh.at[pl.ds(s * RPT, RPT)],
                        deg_out.at[c, pl.ds(s * RPT, RPT)])


_SEG_SEMS = [pltpu.SemaphoreType.DMA] * (NBUF + 2 * NIDX + 2)


def _sc_seg_with_deg(x, src, dst, z2d, zn, ones):
    out_type = (jax.ShapeDtypeStruct((NC, NPAD, D), _f32),
                jax.ShapeDtypeStruct((NC, NPAD), _f32))
    scratch = ([pltpu.VMEM((KC,), _f32)]
               + [pltpu.VMEM((KC,), jnp.int32)] * (2 * NIDX)
               + [pltpu.VMEM((KC, D), _f32)] * NBUF
               + [pltpu.VMEM_SHARED((NPAD, D), _f32),
                  pltpu.VMEM_SHARED((NPAD,), _f32)]
               + _SEG_SEMS + [pltpu.SemaphoreType.DMA] * 2)
    f = pl.kernel(functools.partial(_seg_body, True), out_type=out_type,
                  mesh=_mesh, scratch_types=scratch, compiler_params=_sc_params)
    return f(x, src, dst, z2d, zn, ones)


def _seg_body_nodeg(x_hbm, src_hbm, dst_hbm, z2d_hbm, *refs):
    _seg_body(False, x_hbm, src_hbm, dst_hbm, z2d_hbm, None, None, *refs)


def _sc_seg(x, src, dst, z2d):
    out_type = jax.ShapeDtypeStruct((NC, NPAD, D), _f32)
    scratch = ([pltpu.VMEM((KC,), jnp.int32)] * (2 * NIDX)
               + [pltpu.VMEM((KC, D), _f32)] * NBUF
               + [pltpu.VMEM_SHARED((NPAD, D), _f32)]
               + _SEG_SEMS)
    f = pl.kernel(_seg_body_nodeg, out_type=out_type,
                  mesh=_mesh, scratch_types=scratch, compiler_params=_sc_params)
    return f(x, src, dst, z2d)


_NG = EPW // 16
_ADV_UNROLL = 5


def _adv_body(a_hbm, src_hbm, dst_hbm, zn_hbm, out_hbm,
              a_v, accum_v, src1_v, dst1_v):
    c = lax.axis_index("c")
    s = lax.axis_index("s")
    wid = s * NC + c
    pltpu.sync_copy(a_hbm, a_v)
    pltpu.sync_copy(zn_hbm, accum_v)
    pltpu.sync_copy(src_hbm.at[pl.ds(wid * EPW, EPW)], src1_v)
    pltpu.sync_copy(dst_hbm.at[pl.ds(wid * EPW, EPW)], dst1_v)

    def group(j, carry):
        for v in range(_ADV_UNROLL):
            off = (j * _ADV_UNROLL + v) * 16
            vals = plsc.load_gather(a_v, [src1_v[pl.ds(off, 16)]])
            plsc.addupdate_scatter(accum_v, [dst1_v[pl.ds(off, 16)]], vals)
        return carry

    lax.fori_loop(0, _NG // _ADV_UNROLL, group, 0)
    pltpu.sync_copy(accum_v, out_hbm.at[wid])


def _sc_adv(a, src, dst, zn):
    out_type = jax.ShapeDtypeStruct((NW, N), _f32)
    scratch = [
        pltpu.VMEM((N,), _f32),
        pltpu.VMEM((N,), _f32),
        pltpu.VMEM((EPW,), jnp.int32),
        pltpu.VMEM((EPW,), jnp.int32),
    ]
    f = pl.kernel(_adv_body, out_type=out_type, mesh=_mesh,
                  scratch_types=scratch, compiler_params=_sc_params)
    return f(a, src, dst, zn)


def _dot(a, b):
    return lax.dot_general(a, b, (((1,), (0,)), ((), ())),
                           precision=lax.Precision.HIGHEST,
                           preferred_element_type=_f32)


def _inv_deg(degp):
    deg = jnp.sum(degp.reshape(NC, RB), axis=0)
    return jnp.where(deg > 0, 1.0 / jnp.maximum(deg, 1.0), 0.0)


def _tc1_body(aggp_ref, x_ref, degp_ref, wl_ref, wr_ref, b_ref, h_ref):
    inv = _inv_deg(degp_ref[...])
    mean = (aggp_ref[0] + aggp_ref[1]) * inv[:, None]
    h = _dot(mean, wl_ref[...]) + _dot(x_ref[...], wr_ref[...]) + b_ref[...]
    h_ref[...] = jnp.maximum(h, 0.0)


def _tc1(aggp, x, degp4, wlT, wrT, b):
    return pl.pallas_call(
        _tc1_body,
        grid=(NRB,),
        in_specs=[
            pl.BlockSpec((NC, RB, D), lambda i: (0, i, 0)),
            pl.BlockSpec((RB, D), lambda i: (i, 0)),
            pl.BlockSpec((NC, 1, 1, RB), lambda i: (0, i, 0, 0)),
            pl.BlockSpec((D, D), lambda i: (0, 0)),
            pl.BlockSpec((D, D), lambda i: (0, 0)),
            pl.BlockSpec((1, D), lambda i: (0, 0)),
        ],
        out_specs=pl.BlockSpec((RB, D), lambda i: (i, 0)),
        out_shape=jax.ShapeDtypeStruct((N, D), _f32),
    )(aggp, x, degp4, wlT, wrT, b)


def _one_hot(gi_ref):
    gi = gi_ref[...].reshape(RB)
    return (gi[:, None] == lax.broadcasted_iota(jnp.int32, (RB, G), 1)
            ).astype(_f32)


def _tc2_body(aggp_ref, h_ref, degp_ref, wl_ref, wr_ref, b_ref,
              wa_ref, wr2_ref, gi_ref, a_ref, r_ref, gp_ref):
    i = pl.program_id(0)
    inv = _inv_deg(degp_ref[...])
    mean = (aggp_ref[0] + aggp_ref[1]) * inv[:, None]
    emb = _dot(mean, wl_ref[...]) + _dot(h_ref[...], wr_ref[...]) + b_ref[...]
    a_ref[...] = jnp.sum(emb * wa_ref[...], axis=1).reshape(1, 1, RB)
    r_ref[...] = jnp.sum(emb * wr2_ref[...], axis=1).reshape(1, 1, RB)
    oh = _one_hot(gi_ref)

    @pl.when(i == 0)
    def _():
        gp_ref[...] = jnp.zeros_like(gp_ref)

    gp_ref[...] += lax.dot_general(oh, emb, (((0,), (0,)), ((), ())),
                                   precision=lax.Precision.HIGHEST,
                                   preferred_element_type=_f32)


def _tc2(aggp, h, degp4, wlT, wrT, b, wal, war, gi3):
    return pl.pallas_call(
        _tc2_body,
        grid=(NRB,),
        in_specs=[
            pl.BlockSpec((NC, RB, D), lambda i: (0, i, 0)),
            pl.BlockSpec((RB, D), lambda i: (i, 0)),
            pl.BlockSpec((NC, 1, 1, RB), lambda i: (0, i, 0, 0)),
            pl.BlockSpec((D, D), lambda i: (0, 0)),
            pl.BlockSpec((D, D), lambda i: (0, 0)),
            pl.BlockSpec((1, D), lambda i: (0, 0)),
            pl.BlockSpec((1, D), lambda i: (0, 0)),
            pl.BlockSpec((1, D), lambda i: (0, 0)),
            pl.BlockSpec((1, 1, RB), lambda i: (i, 0, 0)),
        ],
        out_specs=[
            pl.BlockSpec((1, 1, RB), lambda i: (i, 0, 0)),
            pl.BlockSpec((1, 1, RB), lambda i: (i, 0, 0)),
            pl.BlockSpec((G, D), lambda i: (0, 0)),
        ],
        out_shape=[
            jax.ShapeDtypeStruct((NRB, 1, RB), _f32),
            jax.ShapeDtypeStruct((NRB, 1, RB), _f32),
            jax.ShapeDtypeStruct((G, D), _f32),
        ],
    )(aggp, h, degp4, wlT, wrT, b, wal, war, gi3)


def _tc3_body(advp_ref, degp_ref, r_ref, gp_ref, wv_ref, ba_ref, bv_ref,
              gi_ref, out_ref, adv_s, asum_s, cnt_s, val_s):
    p = pl.program_id(0)
    i = pl.program_id(1)
    oh = _one_hot(gi_ref)

    @pl.when(p == 0)
    def _():
        inv = _inv_deg(degp_ref[...])
        agg_a = jnp.sum(advp_ref[...].reshape(NW, RB), axis=0)
        r = r_ref[...].reshape(RB)
        adv = 2.0 * jnp.tanh(agg_a * inv + r + ba_ref[0, 0])
        adv_s[pl.ds(i, 1)] = adv.reshape(1, 1, RB)

        @pl.when(i == 0)
        def _():
            asum_s[...] = jnp.zeros_like(asum_s)
            cnt_s[...] = jnp.zeros_like(cnt_s)
            gp = gp_ref[...]
            val_s[...] = jnp.tanh(
                jnp.sum(gp * wv_ref[...], axis=1) + bv_ref[0, 0]
            ).reshape(1, G)

        asum_s[...] += jnp.sum(oh * adv[:, None], axis=0).reshape(1, G)
        cnt_s[...] += jnp.sum(oh, axis=0).reshape(1, G)

    @pl.when(p == 1)
    def _():
        cnt = cnt_s[...].reshape(G)
        asum = asum_s[...].reshape(G)
        amean = jnp.where(cnt > 0, asum / jnp.maximum(cnt, 1.0), 0.0)
        corr = val_s[...].reshape(G) - amean
        per_row = jnp.sum(oh * corr[None, :], axis=1)
        adv = adv_s[pl.ds(i, 1)].reshape(RB)
        out_ref[...] = jnp.tanh(per_row + adv).reshape(1, 1, RB)


def _tc3(advp4, degp4, r3, gp, wv, ba, bv, gi3):
    return pl.pallas_call(
        _tc3_body,
        grid=(2, NRB),
        in_specs=[
            pl.BlockSpec((NW, 1, 1, RB), lambda p, i: (0, i, 0, 0)),
            pl.BlockSpec((NC, 1, 1, RB), lambda p, i: (0, i, 0, 0)),
            pl.BlockSpec((1, 1, RB), lambda p, i: (i, 0, 0)),
            pl.BlockSpec((G, D), lambda p, i: (0, 0)),
            pl.BlockSpec((1, D), lambda p, i: (0, 0)),
            pl.BlockSpec((1, 1), lambda p, i: (0, 0)),
            pl.BlockSpec((1, 1), lambda p, i: (0, 0)),
            pl.BlockSpec((1, 1, RB), lambda p, i: (i, 0, 0)),
        ],
        out_specs=pl.BlockSpec((1, 1, RB), lambda p, i: (i, 0, 0)),
        out_shape=jax.ShapeDtypeStruct((NRB, 1, RB), _f32),
        scratch_shapes=[
            pltpu.VMEM((NRB, 1, RB), _f32),
            pltpu.VMEM((1, G), _f32),
            pltpu.VMEM((1, G), _f32),
            pltpu.VMEM((1, G), _f32),
        ],
    )(advp4, degp4, r3, gp, wv, ba, bv, gi3)


def kernel(x, edge_index, graph_indices, W1l, W1r, b1, W2l, W2r, b2,
           Wal, War, ba, Wv, bv):
    src = edge_index[0]
    dst = edge_index[1]
    z2d = jnp.zeros((RPT, D), _f32)
    zn = jnp.zeros((N,), _f32)
    ones = jnp.ones((KC,), _f32)
    gi3 = graph_indices.reshape(NRB, 1, RB)

    agg1p, degp = _sc_seg_with_deg(x, src, dst, z2d, zn, ones)
    degp4 = degp[:, :N].reshape(NC, NRB, 1, RB)
    h = _tc1(agg1p, x, degp4, W1l.T, W1r.T, b1.reshape(1, D))
    agg2p = _sc_seg(h, src, dst, z2d)
    a3, r3, gp = _tc2(agg2p, h, degp4, W2l.T, W2r.T, b2.reshape(1, D),
                      Wal, War, gi3)
    advp = _sc_adv(a3.reshape(N), src, dst, zn)
    out3 = _tc3(advp.reshape(NW, NRB, 1, RB), degp4, r3, gp,
                Wv, ba.reshape(1, 1), bv.reshape(1, 1), gi3)
    return out3.reshape(N)
